# baseline (speedup 1.0000x reference)
import functools

import jax
import jax.numpy as jnp
import numpy as np
from jax import lax
from jax.experimental import pallas as pl
from jax.experimental.pallas import tpu as pltpu
from jax.experimental.pallas import tpu_sc as plsc

NUM_FIELDS = 26
N0 = 13
BATCH = 4096
D0 = 64
D1 = 16
NC, NS, L = 2, 16, 16
NW = NC * NS
RPW = BATCH // NW
ROWS_PER_SC = NS * RPW

VOCAB = 1300000
NT = (VOCAB + 127) // 128
TPW = (NT + NW - 1) // NW
S0_ROWS = NT * 64
S1_ROWS = NT * 16

_FIELD_DIMS = np.full(NUM_FIELDS, 100000, dtype=np.int64)
_OFF0 = np.concatenate([[0], np.cumsum(_FIELD_DIMS[:N0])[:-1]]).astype(np.int64)
_OFF1 = np.concatenate([[0], np.cumsum(_FIELD_DIMS[N0:])[:-1]]).astype(np.int64)
_OFFS = [int(v) for v in np.concatenate([_OFF0, _OFF1])]

_mesh = plsc.VectorSubcoreMesh(core_axis_name="c", subcore_axis_name="s")
_params = pltpu.CompilerParams(use_tc_tiling_on_sc=True,
                               needs_layout_passes=False)


@functools.partial(
    pl.kernel,
    mesh=_mesh,
    out_type=(
        jax.ShapeDtypeStruct((S0_ROWS, 128), jnp.float32),
        jax.ShapeDtypeStruct((S1_ROWS, 128), jnp.float32),
    ),
    scratch_types=[
        pltpu.VMEM((2, D0, 128), jnp.float32),
        pltpu.VMEM((2, D1, 128), jnp.float32),
        pltpu.VMEM((2, D0, 128), jnp.float32),
        pltpu.VMEM((2, D1, 128), jnp.float32),
        pltpu.SemaphoreType.DMA((2,)),
        pltpu.SemaphoreType.DMA((2,)),
    ],
    compiler_params=_params,
)
def _relayout_sc(t0t, t1t, s0, s1, in0, in1, ob0, ob1, gin, gout):
    cid = lax.axis_index("c")
    sid = lax.axis_index("s")
    w = cid * NS + sid
    t_base = w * TPW

    def tg_of(i):
        return jnp.minimum(t_base + i, NT - 1)

    def in_copies(i, slot):
        off = pl.multiple_of(tg_of(i) * 128, 128)
        return (
            pltpu.make_async_copy(t0t.at[:, pl.ds(off, 128)], in0.at[slot],
                                  gin.at[slot]),
            pltpu.make_async_copy(t1t.at[:, pl.ds(off, 128)], in1.at[slot],
                                  gin.at[slot]),
        )

    def out_copies(i, slot):
        tg = tg_of(i)
        o0 = pl.multiple_of(tg * D0, D0)
        o1 = pl.multiple_of(tg * D1, D1)
        return (
            pltpu.make_async_copy(ob0.at[slot], s0.at[pl.ds(o0, D0)],
                                  gout.at[slot]),
            pltpu.make_async_copy(ob1.at[slot], s1.at[pl.ds(o1, D1)],
                                  gout.at[slot]),
        )

    IOTA = lax.iota(jnp.int32, L)
    RV0 = [lax.shift_right_logical(IOTA + L * g, 1) for g in range(8)]
    CB0 = [((IOTA + L * g) & 1) * D0 for g in range(8)]
    RV1 = [lax.shift_right_logical(IOTA + L * g, 3) for g in range(8)]
    CB1 = [((IOTA + L * g) & 7) * D1 for g in range(8)]

    def transpose_slab(slot):
        def body0(d, carry):
            for g in range(8):
                plsc.store_scatter(ob0.at[slot], [RV0[g], CB0[g] + d],
                                   in0[slot, d, pl.ds(L * g, L)])
            return carry
        lax.fori_loop(0, D0, body0, 0)

        def body1(d, carry):
            for g in range(8):
                plsc.store_scatter(ob1.at[slot], [RV1[g], CB1[g] + d],
                                   in1[slot, d, pl.ds(L * g, L)])
            return carry
        lax.fori_loop(0, D1, body1, 0)

    for c in in_copies(0, 0):
        c.start()
    for c in in_copies(1, 1):
        c.start()

    def body2(j, carry):
        for slot in (0, 1):
            i = 2 * j + slot
            for c in in_copies(i, slot):
                c.wait()

            @pl.when(j > 0)
            def _():
                for c in out_copies(i - 2, slot):
                    c.wait()

            transpose_slab(slot)
            for c in out_copies(i, slot):
                c.start()

            @pl.when(i + 2 < TPW)
            def _():
                for c in in_copies(i + 2, slot):
                    c.start()
        return carry

    lax.fori_loop(0, TPW // 2, body2, 0)
    for c in out_copies(TPW - 2, 0):
        c.wait()
    for c in out_copies(TPW - 1, 1):
        c.wait()


@functools.partial(
    pl.kernel,
    mesh=_mesh,
    out_type=(
        jax.ShapeDtypeStruct((BATCH, D0), jnp.float32),
        jax.ShapeDtypeStruct((BATCH, D1), jnp.float32),
    ),
    scratch_types=[
        pltpu.VMEM((NUM_FIELDS, RPW), jnp.int32),
        pltpu.VMEM((NUM_FIELDS, RPW), jnp.int32),
        pltpu.VMEM((2, RPW, 128), jnp.float32),
        pltpu.VMEM((2, RPW, 128), jnp.float32),
        pltpu.VMEM((RPW, D0), jnp.float32),
        pltpu.VMEM((RPW, D1), jnp.float32),
        pltpu.SemaphoreType.DMA((2,)),
    ],
    compiler_params=_params,
)
def _pool_sc(xt, s0, s1, out0, out1,
             idx_v, gh_v, gbuf0, gbuf1, ebuf0, ebuf1, gsem):
    cid = lax.axis_index("c")
    sid = lax.axis_index("s")
    wid = cid * NS + sid
    base = wid * RPW

    pltpu.sync_copy(xt.at[:, pl.ds(pl.multiple_of(base, RPW), RPW)], idx_v)

    for g in range(RPW // L):
        sl = pl.ds(g * L, L)
        for f in range(NUM_FIELDS):
            v = idx_v[f, sl] + _OFFS[f]
            idx_v[f, sl] = v
            gh_v[f, sl] = lax.shift_right_logical(v, 1 if f < N0 else 3)

    def start_gather(f):
        slot = f % 2
        if f < N0:
            return pltpu.async_copy(s0.at[gh_v.at[f]], gbuf0.at[slot],
                                    gsem.at[slot])
        return pltpu.async_copy(s1.at[gh_v.at[f]], gbuf1.at[slot],
                                gsem.at[slot])

    IOTA = lax.iota(jnp.int32, L)

    def extract0(f, slot):
        first = f == 0

        def body(g, carry):
            pv = (idx_v[f, pl.ds(g * L, L)] & 1) * D0
            for r0 in range(0, L, 4):
                loaded = []
                for r in range(r0, r0 + 4):
                    b = g * L + r
                    bi = jnp.full((L,), b, jnp.int32)
                    for k in range(D0 // L):
                        loaded.append((b, k, plsc.load_gather(
                            gbuf0.at[slot], [bi, IOTA + (pv[r] + L * k)])))
                for b, k, row in loaded:
                    sl = pl.ds(L * k, L)
                    ebuf0[b, sl] = row if first else ebuf0[b, sl] + row
            return carry
        lax.fori_loop(0, RPW // L, body, 0)

    def extract1(f, slot):
        first = f == N0

        def body(g, carry):
            pv = (idx_v[f, pl.ds(g * L, L)] & 7) * D1
            for r0 in range(0, L, 8):
                loaded = []
                for r in range(r0, r0 + 8):
                    b = g * L + r
                    bi = jnp.full((L,), b, jnp.int32)
                    loaded.append((b, plsc.load_gather(
                        gbuf1.at[slot], [bi, IOTA + pv[r]])))
                for b, row in loaded:
                    ebuf1[b, :] = row if first else ebuf1[b, :] + row
            return carry
        lax.fori_loop(0, RPW // L, body, 0)

    cpy = start_gather(0)
    for f in range(NUM_FIELDS):
        ncpy = start_gather(f + 1) if f + 1 < NUM_FIELDS else None
        cpy.wait()
        slot = f % 2
        if f < N0:
            extract0(f, slot)
        else:
            extract1(f, slot)
        cpy = ncpy

    pltpu.sync_copy(ebuf0, out0.at[pl.ds(base, RPW)])
    pltpu.sync_copy(ebuf1, out1.at[pl.ds(base, RPW)])


def _proj_body(p0_ref, p1_ref, w_ref, b_ref, o_ref):
    proj = lax.dot_general(p1_ref[...], w_ref[...],
                           (((1,), (1,)), ((), ())),
                           preferred_element_type=jnp.float32)
    o_ref[...] = p0_ref[...] + proj + float(N0) * b_ref[...]


_PROJ_GRID = 8
_PB = BATCH // _PROJ_GRID


def _project_tc(pooled0, pooled1, proj_w, proj_b2d):
    return pl.pallas_call(
        _proj_body,
        grid=(_PROJ_GRID,),
        in_specs=[
            pl.BlockSpec((_PB, D0), lambda i: (i, 0)),
            pl.BlockSpec((_PB, D1), lambda i: (i, 0)),
            pl.BlockSpec((D0, D1), lambda i: (0, 0)),
            pl.BlockSpec((1, D0), lambda i: (0, 0)),
        ],
        out_specs=pl.BlockSpec((_PB, D0), lambda i: (i, 0)),
        out_shape=jax.ShapeDtypeStruct((BATCH, D0), jnp.float32),
    )(pooled0, pooled1, proj_w, proj_b2d)


def kernel(x, table0, table1, proj_w, proj_b):
    xt = x.T.astype(jnp.int32)
    s0, s1 = _relayout_sc(table0.T, table1.T)
    pooled0, pooled1 = _pool_sc(xt, s0, s1)
    return _project_tc(pooled0, pooled1, proj_w, proj_b.reshape(1, D0))


# speedup vs baseline: 1.0064x; 1.0064x over previous
import functools

import jax
import jax.numpy as jnp
import numpy as np
from jax import lax
from jax.experimental import pallas as pl
from jax.experimental.pallas import tpu as pltpu
from jax.experimental.pallas import tpu_sc as plsc

NUM_FIELDS = 26
N0 = 13
BATCH = 4096
D0 = 64
D1 = 16
NC, NS, L = 2, 16, 16
NW = NC * NS
RPW = BATCH // NW
ROWS_PER_SC = NS * RPW

VOCAB = 1300000
NT = (VOCAB + 127) // 128
TPW = (NT + NW - 1) // NW
S0_ROWS = NT * 64
S1_ROWS = NT * 16

_FIELD_DIMS = np.full(NUM_FIELDS, 100000, dtype=np.int64)
_OFF0 = np.concatenate([[0], np.cumsum(_FIELD_DIMS[:N0])[:-1]]).astype(np.int64)
_OFF1 = np.concatenate([[0], np.cumsum(_FIELD_DIMS[N0:])[:-1]]).astype(np.int64)
_OFFS = [int(v) for v in np.concatenate([_OFF0, _OFF1])]

_SKIP_TRANSPOSE = False
_mesh = plsc.VectorSubcoreMesh(core_axis_name="c", subcore_axis_name="s")
_params = pltpu.CompilerParams(use_tc_tiling_on_sc=True,
                               needs_layout_passes=False)


@functools.partial(
    pl.kernel,
    mesh=_mesh,
    out_type=(
        jax.ShapeDtypeStruct((S0_ROWS, 128), jnp.float32),
        jax.ShapeDtypeStruct((S1_ROWS, 128), jnp.float32),
    ),
    scratch_types=[
        pltpu.VMEM((2, D0, 128), jnp.float32),
        pltpu.VMEM((2, D1, 128), jnp.float32),
        pltpu.VMEM((2, D0, 128), jnp.float32),
        pltpu.VMEM((2, D1, 128), jnp.float32),
        pltpu.SemaphoreType.DMA((2,)),
        pltpu.SemaphoreType.DMA((2,)),
    ],
    compiler_params=_params,
)
def _relayout_sc(t0t, t1t, s0, s1, in0, in1, ob0, ob1, gin, gout):
    cid = lax.axis_index("c")
    sid = lax.axis_index("s")
    w = cid * NS + sid
    t_base = w * TPW

    def tg_of(i):
        return jnp.minimum(t_base + i, NT - 1)

    def in_copies(i, slot):
        off = pl.multiple_of(tg_of(i) * 128, 128)
        return (
            pltpu.make_async_copy(t0t.at[:, pl.ds(off, 128)], in0.at[slot],
                                  gin.at[slot]),
            pltpu.make_async_copy(t1t.at[:, pl.ds(off, 128)], in1.at[slot],
                                  gin.at[slot]),
        )

    def out_copies(i, slot):
        tg = tg_of(i)
        o0 = pl.multiple_of(tg * D0, D0)
        o1 = pl.multiple_of(tg * D1, D1)
        return (
            pltpu.make_async_copy(ob0.at[slot], s0.at[pl.ds(o0, D0)],
                                  gout.at[slot]),
            pltpu.make_async_copy(ob1.at[slot], s1.at[pl.ds(o1, D1)],
                                  gout.at[slot]),
        )

    IOTA = lax.iota(jnp.int32, L)
    RV0 = [lax.shift_right_logical(IOTA + L * g, 1) for g in range(8)]
    CB0 = [((IOTA + L * g) & 1) * D0 for g in range(8)]
    RV1 = [lax.shift_right_logical(IOTA + L * g, 3) for g in range(8)]
    CB1 = [((IOTA + L * g) & 7) * D1 for g in range(8)]

    def transpose_slab(slot):
        def body0(d, carry):
            vals = [in0[slot, d, pl.ds(L * g, L)] for g in range(8)]
            for g in range(8):
                plsc.store_scatter(ob0.at[slot], [RV0[g], CB0[g] + d],
                                   vals[g])
            return carry
        lax.fori_loop(0, D0, body0, 0)

        def body1(d, carry):
            vals = [in1[slot, d, pl.ds(L * g, L)] for g in range(8)]
            for g in range(8):
                plsc.store_scatter(ob1.at[slot], [RV1[g], CB1[g] + d],
                                   vals[g])
            return carry
        lax.fori_loop(0, D1, body1, 0)

    for c in in_copies(0, 0):
        c.start()
    for c in in_copies(1, 1):
        c.start()

    def body2(j, carry):
        for slot in (0, 1):
            i = 2 * j + slot
            for c in in_copies(i, slot):
                c.wait()

            @pl.when(j > 0)
            def _():
                for c in out_copies(i - 2, slot):
                    c.wait()

            if not _SKIP_TRANSPOSE:
                transpose_slab(slot)
            for c in out_copies(i, slot):
                c.start()

            @pl.when(i + 2 < TPW)
            def _():
                for c in in_copies(i + 2, slot):
                    c.start()
        return carry

    lax.fori_loop(0, TPW // 2, body2, 0)
    for c in out_copies(TPW - 2, 0):
        c.wait()
    for c in out_copies(TPW - 1, 1):
        c.wait()


@functools.partial(
    pl.kernel,
    mesh=_mesh,
    out_type=(
        jax.ShapeDtypeStruct((BATCH, D0), jnp.float32),
        jax.ShapeDtypeStruct((BATCH, D1), jnp.float32),
    ),
    scratch_types=[
        pltpu.VMEM((NUM_FIELDS, RPW), jnp.int32),
        pltpu.VMEM((NUM_FIELDS, RPW), jnp.int32),
        pltpu.VMEM((2, RPW, 128), jnp.float32),
        pltpu.VMEM((2, RPW, 128), jnp.float32),
        pltpu.VMEM((RPW, D0), jnp.float32),
        pltpu.VMEM((RPW, D1), jnp.float32),
        pltpu.SemaphoreType.DMA((2,)),
    ],
    compiler_params=_params,
)
def _pool_sc(xt, s0, s1, out0, out1,
             idx_v, gh_v, gbuf0, gbuf1, ebuf0, ebuf1, gsem):
    cid = lax.axis_index("c")
    sid = lax.axis_index("s")
    wid = cid * NS + sid
    base = wid * RPW

    pltpu.sync_copy(xt.at[:, pl.ds(pl.multiple_of(base, RPW), RPW)], idx_v)

    for g in range(RPW // L):
        sl = pl.ds(g * L, L)
        for f in range(NUM_FIELDS):
            v = idx_v[f, sl] + _OFFS[f]
            idx_v[f, sl] = v
            gh_v[f, sl] = lax.shift_right_logical(v, 1 if f < N0 else 3)

    def start_gather(f):
        slot = f % 2
        if f < N0:
            return pltpu.async_copy(s0.at[gh_v.at[f]], gbuf0.at[slot],
                                    gsem.at[slot])
        return pltpu.async_copy(s1.at[gh_v.at[f]], gbuf1.at[slot],
                                gsem.at[slot])

    IOTA = lax.iota(jnp.int32, L)

    def extract0(f, slot):
        first = f == 0

        def body(g, carry):
            pv = (idx_v[f, pl.ds(g * L, L)] & 1) * D0
            for r0 in range(0, L, 4):
                loaded = []
                for r in range(r0, r0 + 4):
                    b = g * L + r
                    bi = jnp.full((L,), b, jnp.int32)
                    for k in range(D0 // L):
                        loaded.append((b, k, plsc.load_gather(
                            gbuf0.at[slot], [bi, IOTA + (pv[r] + L * k)])))
                for b, k, row in loaded:
                    sl = pl.ds(L * k, L)
                    ebuf0[b, sl] = row if first else ebuf0[b, sl] + row
            return carry
        lax.fori_loop(0, RPW // L, body, 0)

    def extract1(f, slot):
        first = f == N0

        def body(g, carry):
            pv = (idx_v[f, pl.ds(g * L, L)] & 7) * D1
            for r0 in range(0, L, 8):
                loaded = []
                for r in range(r0, r0 + 8):
                    b = g * L + r
                    bi = jnp.full((L,), b, jnp.int32)
                    loaded.append((b, plsc.load_gather(
                        gbuf1.at[slot], [bi, IOTA + pv[r]])))
                for b, row in loaded:
                    ebuf1[b, :] = row if first else ebuf1[b, :] + row
            return carry
        lax.fori_loop(0, RPW // L, body, 0)

    cpy = start_gather(0)
    for f in range(NUM_FIELDS):
        ncpy = start_gather(f + 1) if f + 1 < NUM_FIELDS else None
        cpy.wait()
        slot = f % 2
        if f < N0:
            extract0(f, slot)
        else:
            extract1(f, slot)
        cpy = ncpy

    pltpu.sync_copy(ebuf0, out0.at[pl.ds(base, RPW)])
    pltpu.sync_copy(ebuf1, out1.at[pl.ds(base, RPW)])


def _proj_body(p0_ref, p1_ref, w_ref, b_ref, o_ref):
    proj = lax.dot_general(p1_ref[...], w_ref[...],
                           (((1,), (1,)), ((), ())),
                           preferred_element_type=jnp.float32)
    o_ref[...] = p0_ref[...] + proj + float(N0) * b_ref[...]


_PROJ_GRID = 8
_PB = BATCH // _PROJ_GRID


def _project_tc(pooled0, pooled1, proj_w, proj_b2d):
    return pl.pallas_call(
        _proj_body,
        grid=(_PROJ_GRID,),
        in_specs=[
            pl.BlockSpec((_PB, D0), lambda i: (i, 0)),
            pl.BlockSpec((_PB, D1), lambda i: (i, 0)),
            pl.BlockSpec((D0, D1), lambda i: (0, 0)),
            pl.BlockSpec((1, D0), lambda i: (0, 0)),
        ],
        out_specs=pl.BlockSpec((_PB, D0), lambda i: (i, 0)),
        out_shape=jax.ShapeDtypeStruct((BATCH, D0), jnp.float32),
    )(pooled0, pooled1, proj_w, proj_b2d)


def kernel(x, table0, table1, proj_w, proj_b):
    xt = x.T.astype(jnp.int32)
    s0, s1 = _relayout_sc(table0.T, table1.T)
    pooled0, pooled1 = _pool_sc(xt, s0, s1)
    return _project_tc(pooled0, pooled1, proj_w, proj_b.reshape(1, D0))


# speedup vs baseline: 1.4834x; 1.4741x over previous
import functools

import jax
import jax.numpy as jnp
import numpy as np
from jax import lax
from jax.experimental import pallas as pl
from jax.experimental.pallas import tpu as pltpu
from jax.experimental.pallas import tpu_sc as plsc

NUM_FIELDS = 26
N0 = 13
BATCH = 4096
D0 = 64
D1 = 16
NC, NS, L = 2, 16, 16
NW = NC * NS
RPW = BATCH // NW
ROWS_PER_SC = NS * RPW

_FIELD_DIMS = np.full(NUM_FIELDS, 100000, dtype=np.int64)
_OFF0 = np.concatenate([[0], np.cumsum(_FIELD_DIMS[:N0])[:-1]]).astype(np.int64)
_OFF1 = np.concatenate([[0], np.cumsum(_FIELD_DIMS[N0:])[:-1]]).astype(np.int64)
_OFFS = [int(v) for v in np.concatenate([_OFF0, _OFF1])]

_mesh = plsc.VectorSubcoreMesh(core_axis_name="c", subcore_axis_name="s")


@functools.partial(
    pl.kernel,
    mesh=_mesh,
    out_type=(
        jax.ShapeDtypeStruct((BATCH, D0), jnp.float32),
        jax.ShapeDtypeStruct((BATCH, D1), jnp.float32),
    ),
    scratch_types=[
        pltpu.VMEM((NUM_FIELDS, RPW), jnp.int32),
        pltpu.VMEM((RPW,), jnp.int32),
        pltpu.VMEM((2, RPW, D0), jnp.float32),
        pltpu.VMEM((2, RPW, D1), jnp.float32),
        pltpu.VMEM_SHARED((ROWS_PER_SC, D0), jnp.float32),
        pltpu.VMEM_SHARED((ROWS_PER_SC, D1), jnp.float32),
        pltpu.SemaphoreType.DMA((2,)),
    ],
    compiler_params=pltpu.CompilerParams(use_tc_tiling_on_sc=False),
)
def _pool_sc(xt, table0, table1, out0, out1,
             idx_v, rid_v, buf0, buf1, acc0, acc1, gsem):
    cid = lax.axis_index("c")
    sid = lax.axis_index("s")
    wid = cid * NS + sid
    base = wid * RPW
    sbase = sid * RPW

    pltpu.sync_copy(xt.at[:, pl.ds(base, RPW)], idx_v)

    for g in range(RPW // L):
        sl = pl.ds(g * L, L)
        rid_v[sl] = lax.iota(jnp.int32, L) + (sbase + g * L)
        for f in range(NUM_FIELDS):
            if _OFFS[f]:
                idx_v[f, sl] = idx_v[f, sl] + _OFFS[f]

    def start_gather(f):
        slot = f % 2
        if f < N0:
            return pltpu.async_copy(table0.at[idx_v.at[f]], buf0.at[slot],
                                    gsem.at[slot])
        return pltpu.async_copy(table1.at[idx_v.at[f]], buf1.at[slot],
                                gsem.at[slot])

    cpy = start_gather(0)
    for f in range(NUM_FIELDS):
        ncpy = start_gather(f + 1) if f + 1 < NUM_FIELDS else None
        cpy.wait()
        slot = f % 2
        if f < N0:
            if f == 0:
                pltpu.sync_copy(buf0.at[slot], acc0.at[pl.ds(sbase, RPW)])
            else:
                pltpu.sync_copy(buf0.at[slot], acc0.at[rid_v], add=True)
        else:
            if f == N0:
                pltpu.sync_copy(buf1.at[slot], acc1.at[pl.ds(sbase, RPW)])
            else:
                pltpu.sync_copy(buf1.at[slot], acc1.at[rid_v], add=True)
        cpy = ncpy

    pltpu.sync_copy(acc0.at[pl.ds(sbase, RPW)], out0.at[pl.ds(base, RPW)])
    pltpu.sync_copy(acc1.at[pl.ds(sbase, RPW)], out1.at[pl.ds(base, RPW)])


def _proj_body(p0_ref, p1_ref, w_ref, b_ref, o_ref):
    proj = lax.dot_general(p1_ref[...], w_ref[...],
                           (((1,), (1,)), ((), ())),
                           preferred_element_type=jnp.float32)
    o_ref[...] = p0_ref[...] + proj + float(N0) * b_ref[...]


_PROJ_GRID = 8
_PB = BATCH // _PROJ_GRID


def _project_tc(pooled0, pooled1, proj_w, proj_b2d):
    return pl.pallas_call(
        _proj_body,
        grid=(_PROJ_GRID,),
        in_specs=[
            pl.BlockSpec((_PB, D0), lambda i: (i, 0)),
            pl.BlockSpec((_PB, D1), lambda i: (i, 0)),
            pl.BlockSpec((D0, D1), lambda i: (0, 0)),
            pl.BlockSpec((1, D0), lambda i: (0, 0)),
        ],
        out_specs=pl.BlockSpec((_PB, D0), lambda i: (i, 0)),
        out_shape=jax.ShapeDtypeStruct((BATCH, D0), jnp.float32),
    )(pooled0, pooled1, proj_w, proj_b2d)


def kernel(x, table0, table1, proj_w, proj_b):
    xt = x.T.astype(jnp.int32)
    pooled0, pooled1 = _pool_sc(xt, table0, table1)
    return _project_tc(pooled0, pooled1, proj_w, proj_b.reshape(1, D0))
